# baseline (speedup 1.0000x reference)
import functools

import jax
import jax.numpy as jnp
from jax import lax
from jax.experimental import pallas as pl
from jax.experimental.pallas import tpu as pltpu
from jax.experimental.pallas import tpu_sc as plsc

_N = 10000
_D = 128
_NTILES = 32
_STRIPE = 640
_ACC_ROWS = _NTILES // 2 * _STRIPE
_CHUNK = 128
_BLKC = 48
_LANES = 16


def _seg_accum(table, idxs, plane, nc0, nc1):
  mesh = plsc.VectorSubcoreMesh(core_axis_name="c", subcore_axis_name="s")

  def body(table_hbm, idx_hbm, out_hbm, idx_v, dst_v, rows0, rows1,
           acc, sem0, sem1, ssem0, ssem1):
    cid = lax.axis_index("c")
    sid = lax.axis_index("s")
    wid = cid * 16 + sid

    def zrow(i, c):
      r = i // (_D // _LANES)
      k = i % (_D // _LANES)
      rows0[r, pl.ds(k * _LANES, _LANES)] = jnp.zeros((_LANES,), jnp.float32)
      return c
    lax.fori_loop(0, _CHUNK * (_D // _LANES), zrow, 0)
    for j in range(_STRIPE // _CHUNK):
      pltpu.sync_copy(rows0, acc.at[pl.ds(sid * _STRIPE + j * _CHUNK, _CHUNK)])

    plsc.subcore_barrier()

    nc_mine = jnp.where(cid == 0, nc0, nc1)
    nblocks = (nc_mine + _BLKC - 1) // _BLKC

    def gather(j, buf, sem):
      pltpu.async_copy(table_hbm.at[idx_v.at[j]], buf, sem)

    def gather_wait(j, buf, sem):
      pltpu.make_async_copy(table_hbm.at[idx_v.at[j]], buf, sem).wait()

    def scatter(j, buf, sem):
      pltpu.async_copy(buf, acc.at[dst_v.at[j]], sem, add=True)

    def scatter_wait(j, buf, sem):
      pltpu.make_async_copy(buf, acc.at[dst_v.at[j]], sem).wait()

    def block(b, c):
      base = b * _BLKC
      cnt = jnp.minimum(_BLKC, nc_mine - base)
      pltpu.sync_copy(idx_hbm.at[plane, wid, pl.ds(base, _BLKC)], idx_v)
      pltpu.sync_copy(idx_hbm.at[1, wid, pl.ds(base, _BLKC)], dst_v)
      gather(0, rows0, sem0)
      gather(1, rows1, sem1)

      def step2(i, c2):
        j = i * 2
        gather_wait(j, rows0, sem0)
        scatter(j, rows0, ssem0)
        gather_wait(j + 1, rows1, sem1)
        scatter(j + 1, rows1, ssem1)
        scatter_wait(j, rows0, ssem0)
        gather(j + 2, rows0, sem0)
        scatter_wait(j + 1, rows1, ssem1)
        gather(j + 3, rows1, sem1)
        return c2

      lax.fori_loop(0, cnt // 2 - 1, step2, 0)

      jl = cnt - 2
      gather_wait(jl, rows0, sem0)
      scatter(jl, rows0, ssem0)
      gather_wait(jl + 1, rows1, sem1)
      scatter(jl + 1, rows1, ssem1)
      scatter_wait(jl, rows0, ssem0)
      scatter_wait(jl + 1, rows1, ssem1)
      return c

    lax.fori_loop(0, nblocks, block, 0)

    plsc.subcore_barrier()
    pltpu.sync_copy(acc.at[pl.ds(sid * _STRIPE, _STRIPE)],
                    out_hbm.at[cid, pl.ds(sid * _STRIPE, _STRIPE)])

  k = pl.kernel(
      body,
      out_type=jax.ShapeDtypeStruct((2, _ACC_ROWS, _D), jnp.float32),
      mesh=mesh,
      scratch_types=[
          pltpu.VMEM((_BLKC, _CHUNK), jnp.int32),
          pltpu.VMEM((_BLKC, _CHUNK), jnp.int32),
          pltpu.VMEM((_CHUNK, _D), jnp.float32),
          pltpu.VMEM((_CHUNK, _D), jnp.float32),
          pltpu.VMEM_SHARED((_ACC_ROWS, _D), jnp.float32),
          pltpu.SemaphoreType.DMA,
          pltpu.SemaphoreType.DMA,
          pltpu.SemaphoreType.DMA,
          pltpu.SemaphoreType.DMA,
      ],
  )
  return k(table, idxs)


_BLK = 400

_mm = functools.partial(jnp.dot, preferred_element_type=jnp.float32,
                        precision=lax.Precision.HIGHEST)


def _gates_body(p1, p2, p3, h, wz, wr, bz, br, z_o, m_o, f_o, t1a_o):
  t1a = p1[0] + p1[1]
  t1b = p2[0] + p2[1]
  f = p3[0] + p3[1]
  hh = h[...]
  z = jax.nn.sigmoid(_mm(t1a, wz[:_D]) + _mm(t1b, wz[_D:]) + f + bz[...])
  r = jax.nn.sigmoid(_mm(t1a, wr[:_D]) + _mm(t1b, wr[_D:]) + f + br[...])
  z_o[...] = z
  m_o[...] = r * hh
  f_o[...] = f
  t1a_o[...] = t1a


def _gates(P1, P2, P3, H_prev, W_z, W_r, b_z, b_r):
  part = pl.BlockSpec((2, _BLK, _D), lambda i: (0, i, 0))
  node = pl.BlockSpec((_BLK, _D), lambda i: (i, 0))
  wspec = pl.BlockSpec((2 * _D, _D), lambda i: (0, 0))
  bspec = pl.BlockSpec((1, _D), lambda i: (0, 0))
  out = jax.ShapeDtypeStruct((_N, _D), jnp.float32)
  return pl.pallas_call(
      _gates_body,
      grid=(_N // _BLK,),
      in_specs=[part, part, part, node, wspec, wspec, bspec, bspec],
      out_specs=[node, node, node, node],
      out_shape=[out, out, out, out],
  )(P1, P2, P3, H_prev, W_z, W_r, b_z, b_r)


def _update_body(t1a, p4, f, z, h, wh, bh, h_o):
  t2 = p4[0] + p4[1]
  g = _mm(t1a[...], wh[:_D]) + _mm(t2, wh[_D:]) + f[...] + bh[...]
  h_tilde = jnp.tanh(g)
  zz = z[...]
  h_o[...] = zz * h[...] + (1.0 - zz) * h_tilde


def _update(T1a, P4, F, Z, H_prev, W_h, b_h):
  part = pl.BlockSpec((2, _BLK, _D), lambda i: (0, i, 0))
  node = pl.BlockSpec((_BLK, _D), lambda i: (i, 0))
  wspec = pl.BlockSpec((2 * _D, _D), lambda i: (0, 0))
  bspec = pl.BlockSpec((1, _D), lambda i: (0, 0))
  return pl.pallas_call(
      _update_body,
      grid=(_N // _BLK,),
      in_specs=[node, part, node, node, node, wspec, bspec],
      out_specs=node,
      out_shape=jax.ShapeDtypeStruct((_N, _D), jnp.float32),
  )(T1a, P4, F, Z, H_prev, W_h, b_h)


def _split_chunks(e):
  total = -(-e // (16 * _CHUNK))
  nc0 = -(-total // 2)
  nc0 += nc0 % 2
  nc1 = max(total - nc0, 2)
  nc1 += nc1 % 2
  return nc0, nc1


def _pack(edge_index, e, nc0, nc1):
  e0 = 16 * nc0 * _CHUNK
  e1 = 16 * nc1 * _CHUNK
  ncp = -(-max(nc0, nc1) // _BLKC) * _BLKC
  ppos = jnp.arange(e0 + e1 - e, dtype=jnp.int32)
  arrs = jnp.stack([edge_index[0], edge_index[1],
                    jnp.arange(e, dtype=jnp.int32)])
  pads = jnp.stack([ppos % _N, _N + ppos % (_ACC_ROWS - _N), ppos % e])
  a = jnp.concatenate([arrs, pads], axis=1)
  a0 = jnp.pad(a[:, :e0].reshape(3, 16, nc0, _CHUNK),
               ((0, 0), (0, 0), (0, ncp - nc0), (0, 0)))
  a1 = jnp.pad(a[:, e0:].reshape(3, 16, nc1, _CHUNK),
               ((0, 0), (0, 0), (0, ncp - nc1), (0, 0)))
  return jnp.concatenate([a0, a1], axis=1)


def kernel(X, H_prev, edge_index, feat, W_z, b_z, W_r, b_r, W_h, b_h):
  e = edge_index.shape[1]
  nc0, nc1 = _split_chunks(e)

  idxs = _pack(edge_index, e, nc0, nc1)

  P1 = _seg_accum(X, idxs, 0, nc0, nc1)
  P2 = _seg_accum(H_prev, idxs, 0, nc0, nc1)
  P3 = _seg_accum(feat, idxs, 2, nc0, nc1)

  Z, M, F, T1a = _gates(P1, P2, P3, H_prev,
                        W_z, W_r, b_z.reshape(1, _D), b_r.reshape(1, _D))

  P4 = _seg_accum(M, idxs, 0, nc0, nc1)

  return _update(T1a, P4, F, Z, H_prev, W_h, b_h.reshape(1, _D))


# speedup vs baseline: 1.1059x; 1.1059x over previous
import functools

import jax
import jax.numpy as jnp
from jax import lax
from jax.experimental import pallas as pl
from jax.experimental.pallas import tpu as pltpu
from jax.experimental.pallas import tpu_sc as plsc

_N = 10000
_D = 128
_NTILES = 32
_STRIPE = 640
_ACC_ROWS = _NTILES // 2 * _STRIPE
_CHUNK = 128
_BLKC = 48
_LANES = 16


def _seg_accum(table, idxs, plane, nc0, nc1):
  mesh = plsc.VectorSubcoreMesh(core_axis_name="c", subcore_axis_name="s")

  def body(table_hbm, idx_hbm, out_hbm, idx_v, dst_v, rows0, rows1,
           acc, sem0, sem1, ssem0, ssem1):
    cid = lax.axis_index("c")
    sid = lax.axis_index("s")
    wid = cid * 16 + sid

    def zrow(i, c):
      r = i // (_D // _LANES)
      k = i % (_D // _LANES)
      rows0[r, pl.ds(k * _LANES, _LANES)] = jnp.zeros((_LANES,), jnp.float32)
      return c
    lax.fori_loop(0, _CHUNK * (_D // _LANES), zrow, 0)
    for j in range(_STRIPE // _CHUNK):
      pltpu.sync_copy(rows0, acc.at[pl.ds(sid * _STRIPE + j * _CHUNK, _CHUNK)])

    plsc.subcore_barrier()

    nc_mine = jnp.where(cid == 0, nc0, nc1)
    nblocks = (nc_mine + _BLKC - 1) // _BLKC

    def gather(j, buf, sem):
      pltpu.async_copy(table_hbm.at[idx_v.at[j]], buf, sem)

    def gather_wait(j, buf, sem):
      pltpu.make_async_copy(table_hbm.at[idx_v.at[j]], buf, sem).wait()

    def scatter(j, buf, sem):
      pltpu.async_copy(buf, acc.at[dst_v.at[j]], sem, add=True)

    def scatter_wait(j, buf, sem):
      pltpu.make_async_copy(buf, acc.at[dst_v.at[j]], sem).wait()

    def block(b, c):
      base = b * _BLKC
      cnt = jnp.minimum(_BLKC, nc_mine - base)
      pltpu.sync_copy(idx_hbm.at[plane, wid, pl.ds(base, _BLKC)], idx_v)
      pltpu.sync_copy(idx_hbm.at[1, wid, pl.ds(base, _BLKC)], dst_v)
      gather(0, rows0, sem0)

      def step2(i, c2):
        j = i * 2
        gather_wait(j, rows0, sem0)
        gather(j + 1, rows1, sem1)
        pltpu.sync_copy(rows0, acc.at[dst_v.at[j]], add=True)

        gather_wait(j + 1, rows1, sem1)
        gather(j + 2, rows0, sem0)
        pltpu.sync_copy(rows1, acc.at[dst_v.at[j + 1]], add=True)
        return c2

      lax.fori_loop(0, cnt // 2 - 1, step2, 0)

      jl = cnt - 2
      gather_wait(jl, rows0, sem0)
      gather(jl + 1, rows1, sem1)
      pltpu.sync_copy(rows0, acc.at[dst_v.at[jl]], add=True)
      gather_wait(jl + 1, rows1, sem1)
      pltpu.sync_copy(rows1, acc.at[dst_v.at[jl + 1]], add=True)
      return c

    lax.fori_loop(0, nblocks, block, 0)

    plsc.subcore_barrier()
    pltpu.sync_copy(acc.at[pl.ds(sid * _STRIPE, _STRIPE)],
                    out_hbm.at[cid, pl.ds(sid * _STRIPE, _STRIPE)])

  k = pl.kernel(
      body,
      out_type=jax.ShapeDtypeStruct((2, _ACC_ROWS, _D), jnp.float32),
      mesh=mesh,
      scratch_types=[
          pltpu.VMEM((_BLKC, _CHUNK), jnp.int32),
          pltpu.VMEM((_BLKC, _CHUNK), jnp.int32),
          pltpu.VMEM((_CHUNK, _D), jnp.float32),
          pltpu.VMEM((_CHUNK, _D), jnp.float32),
          pltpu.VMEM_SHARED((_ACC_ROWS, _D), jnp.float32),
          pltpu.SemaphoreType.DMA,
          pltpu.SemaphoreType.DMA,
          pltpu.SemaphoreType.DMA,
          pltpu.SemaphoreType.DMA,
      ],
  )
  return k(table, idxs)


_BLK = 400

_mm = functools.partial(jnp.dot, preferred_element_type=jnp.float32,
                        precision=lax.Precision.HIGHEST)


def _gates_body(p1, p2, p3, h, wz, wr, bz, br, z_o, m_o, f_o, t1a_o):
  t1a = p1[0] + p1[1]
  t1b = p2[0] + p2[1]
  f = p3[0] + p3[1]
  hh = h[...]
  z = jax.nn.sigmoid(_mm(t1a, wz[:_D]) + _mm(t1b, wz[_D:]) + f + bz[...])
  r = jax.nn.sigmoid(_mm(t1a, wr[:_D]) + _mm(t1b, wr[_D:]) + f + br[...])
  z_o[...] = z
  m_o[...] = r * hh
  f_o[...] = f
  t1a_o[...] = t1a


def _gates(P1, P2, P3, H_prev, W_z, W_r, b_z, b_r):
  part = pl.BlockSpec((2, _BLK, _D), lambda i: (0, i, 0))
  node = pl.BlockSpec((_BLK, _D), lambda i: (i, 0))
  wspec = pl.BlockSpec((2 * _D, _D), lambda i: (0, 0))
  bspec = pl.BlockSpec((1, _D), lambda i: (0, 0))
  out = jax.ShapeDtypeStruct((_N, _D), jnp.float32)
  return pl.pallas_call(
      _gates_body,
      grid=(_N // _BLK,),
      in_specs=[part, part, part, node, wspec, wspec, bspec, bspec],
      out_specs=[node, node, node, node],
      out_shape=[out, out, out, out],
  )(P1, P2, P3, H_prev, W_z, W_r, b_z, b_r)


def _update_body(t1a, p4, f, z, h, wh, bh, h_o):
  t2 = p4[0] + p4[1]
  g = _mm(t1a[...], wh[:_D]) + _mm(t2, wh[_D:]) + f[...] + bh[...]
  h_tilde = jnp.tanh(g)
  zz = z[...]
  h_o[...] = zz * h[...] + (1.0 - zz) * h_tilde


def _update(T1a, P4, F, Z, H_prev, W_h, b_h):
  part = pl.BlockSpec((2, _BLK, _D), lambda i: (0, i, 0))
  node = pl.BlockSpec((_BLK, _D), lambda i: (i, 0))
  wspec = pl.BlockSpec((2 * _D, _D), lambda i: (0, 0))
  bspec = pl.BlockSpec((1, _D), lambda i: (0, 0))
  return pl.pallas_call(
      _update_body,
      grid=(_N // _BLK,),
      in_specs=[node, part, node, node, node, wspec, bspec],
      out_specs=node,
      out_shape=jax.ShapeDtypeStruct((_N, _D), jnp.float32),
  )(T1a, P4, F, Z, H_prev, W_h, b_h)


def _split_chunks(e):
  total = -(-e // (16 * _CHUNK))
  nc0 = -(-total // 2)
  nc0 += nc0 % 2
  nc1 = max(total - nc0, 2)
  nc1 += nc1 % 2
  return nc0, nc1


def _pack(edge_index, e, nc0, nc1):
  e0 = 16 * nc0 * _CHUNK
  e1 = 16 * nc1 * _CHUNK
  ncp = -(-max(nc0, nc1) // _BLKC) * _BLKC
  ppos = jnp.arange(e0 + e1 - e, dtype=jnp.int32)
  arrs = jnp.stack([edge_index[0], edge_index[1],
                    jnp.arange(e, dtype=jnp.int32)])
  pads = jnp.stack([ppos % _N, _N + ppos % (_ACC_ROWS - _N), ppos % e])
  a = jnp.concatenate([arrs, pads], axis=1)
  a0 = jnp.pad(a[:, :e0].reshape(3, 16, nc0, _CHUNK),
               ((0, 0), (0, 0), (0, ncp - nc0), (0, 0)))
  a1 = jnp.pad(a[:, e0:].reshape(3, 16, nc1, _CHUNK),
               ((0, 0), (0, 0), (0, ncp - nc1), (0, 0)))
  return jnp.concatenate([a0, a1], axis=1)


def kernel(X, H_prev, edge_index, feat, W_z, b_z, W_r, b_r, W_h, b_h):
  e = edge_index.shape[1]
  nc0, nc1 = _split_chunks(e)

  idxs = _pack(edge_index, e, nc0, nc1)

  P1 = _seg_accum(X, idxs, 0, nc0, nc1)
  P2 = _seg_accum(H_prev, idxs, 0, nc0, nc1)
  P3 = _seg_accum(feat, idxs, 2, nc0, nc1)

  Z, M, F, T1a = _gates(P1, P2, P3, H_prev,
                        W_z, W_r, b_z.reshape(1, _D), b_r.reshape(1, _D))

  P4 = _seg_accum(M, idxs, 0, nc0, nc1)

  return _update(T1a, P4, F, Z, H_prev, W_h, b_h.reshape(1, _D))


# speedup vs baseline: 1.1142x; 1.0075x over previous
import functools

import jax
import jax.numpy as jnp
from jax import lax
from jax.experimental import pallas as pl
from jax.experimental.pallas import tpu as pltpu
from jax.experimental.pallas import tpu_sc as plsc

_N = 10000
_D = 128
_NTILES = 32
_STRIPE = 640
_ACC_ROWS = _NTILES // 2 * _STRIPE
_CHUNK = 128
_BLKC = 48
_LANES = 16


def _seg_accum(table, idxs, plane, nc0, nc1):
  mesh = plsc.VectorSubcoreMesh(core_axis_name="c", subcore_axis_name="s")

  def body(table_hbm, idx_hbm, out_hbm, idx_v, dst_v, rows0, rows1,
           acc, sem0, sem1, ssem0, ssem1):
    cid = lax.axis_index("c")
    sid = lax.axis_index("s")
    wid = cid * 16 + sid

    def zrow(i, c):
      r = i // (_D // _LANES)
      k = i % (_D // _LANES)
      rows0[r, pl.ds(k * _LANES, _LANES)] = jnp.zeros((_LANES,), jnp.float32)
      return c
    lax.fori_loop(0, _CHUNK * (_D // _LANES), zrow, 0)
    for j in range(_STRIPE // _CHUNK):
      pltpu.sync_copy(rows0, acc.at[pl.ds(sid * _STRIPE + j * _CHUNK, _CHUNK)])

    plsc.subcore_barrier()

    nc_mine = jnp.where(cid == 0, nc0, nc1)
    nblocks = (nc_mine + _BLKC - 1) // _BLKC

    def gather(j, buf, sem):
      pltpu.async_copy(table_hbm.at[idx_v.at[j]], buf, sem)

    def gather_wait(j, buf, sem):
      pltpu.make_async_copy(table_hbm.at[idx_v.at[j]], buf, sem).wait()

    def scatter(j, buf, sem):
      pltpu.async_copy(buf, acc.at[dst_v.at[j]], sem, add=True)

    def scatter_wait(j, buf, sem):
      pltpu.make_async_copy(buf, acc.at[dst_v.at[j]], sem).wait()

    def block(b, c):
      base = b * _BLKC
      cnt = jnp.minimum(_BLKC, nc_mine - base)
      pltpu.sync_copy(idx_hbm.at[plane, wid, pl.ds(base, _BLKC)], idx_v)
      pltpu.sync_copy(idx_hbm.at[1, wid, pl.ds(base, _BLKC)], dst_v)
      gather(0, rows0, sem0)

      def step2(i, c2):
        j = i * 2
        gather_wait(j, rows0, sem0)
        gather(j + 1, rows1, sem1)
        pltpu.sync_copy(rows0, acc.at[dst_v.at[j]], add=True)

        gather_wait(j + 1, rows1, sem1)
        gather(j + 2, rows0, sem0)
        pltpu.sync_copy(rows1, acc.at[dst_v.at[j + 1]], add=True)
        return c2

      lax.fori_loop(0, cnt // 2 - 1, step2, 0)

      jl = cnt - 2
      gather_wait(jl, rows0, sem0)
      gather(jl + 1, rows1, sem1)
      pltpu.sync_copy(rows0, acc.at[dst_v.at[jl]], add=True)
      gather_wait(jl + 1, rows1, sem1)
      pltpu.sync_copy(rows1, acc.at[dst_v.at[jl + 1]], add=True)
      return c

    lax.fori_loop(0, nblocks, block, 0)

    plsc.subcore_barrier()
    pltpu.sync_copy(acc.at[pl.ds(sid * _STRIPE, _STRIPE)],
                    out_hbm.at[cid, pl.ds(sid * _STRIPE, _STRIPE)])

  k = pl.kernel(
      body,
      out_type=jax.ShapeDtypeStruct((2, _ACC_ROWS, _D), jnp.float32),
      mesh=mesh,
      scratch_types=[
          pltpu.VMEM((_BLKC, _CHUNK), jnp.int32),
          pltpu.VMEM((_BLKC, _CHUNK), jnp.int32),
          pltpu.VMEM((_CHUNK, _D), jnp.float32),
          pltpu.VMEM((_CHUNK, _D), jnp.float32),
          pltpu.VMEM_SHARED((_ACC_ROWS, _D), jnp.float32),
          pltpu.SemaphoreType.DMA,
          pltpu.SemaphoreType.DMA,
          pltpu.SemaphoreType.DMA,
          pltpu.SemaphoreType.DMA,
      ],
  )
  return k(table, idxs)


_BLK = 1000

_mm = functools.partial(jnp.dot, preferred_element_type=jnp.float32,
                        precision=lax.Precision.HIGHEST)


def _gates_body(p1, p2, p3, h, wz, wr, bz, br, z_o, m_o, f_o, t1a_o):
  t1a = p1[0] + p1[1]
  t1b = p2[0] + p2[1]
  f = p3[0] + p3[1]
  hh = h[...]
  z = jax.nn.sigmoid(_mm(t1a, wz[:_D]) + _mm(t1b, wz[_D:]) + f + bz[...])
  r = jax.nn.sigmoid(_mm(t1a, wr[:_D]) + _mm(t1b, wr[_D:]) + f + br[...])
  z_o[...] = z
  m_o[...] = r * hh
  f_o[...] = f
  t1a_o[...] = t1a


def _gates(P1, P2, P3, H_prev, W_z, W_r, b_z, b_r):
  part = pl.BlockSpec((2, _BLK, _D), lambda i: (0, i, 0))
  node = pl.BlockSpec((_BLK, _D), lambda i: (i, 0))
  wspec = pl.BlockSpec((2 * _D, _D), lambda i: (0, 0))
  bspec = pl.BlockSpec((1, _D), lambda i: (0, 0))
  out = jax.ShapeDtypeStruct((_N, _D), jnp.float32)
  return pl.pallas_call(
      _gates_body,
      grid=(_N // _BLK,),
      in_specs=[part, part, part, node, wspec, wspec, bspec, bspec],
      out_specs=[node, node, node, node],
      out_shape=[out, out, out, out],
  )(P1, P2, P3, H_prev, W_z, W_r, b_z, b_r)


def _update_body(t1a, p4, f, z, h, wh, bh, h_o):
  t2 = p4[0] + p4[1]
  g = _mm(t1a[...], wh[:_D]) + _mm(t2, wh[_D:]) + f[...] + bh[...]
  h_tilde = jnp.tanh(g)
  zz = z[...]
  h_o[...] = zz * h[...] + (1.0 - zz) * h_tilde


def _update(T1a, P4, F, Z, H_prev, W_h, b_h):
  part = pl.BlockSpec((2, _BLK, _D), lambda i: (0, i, 0))
  node = pl.BlockSpec((_BLK, _D), lambda i: (i, 0))
  wspec = pl.BlockSpec((2 * _D, _D), lambda i: (0, 0))
  bspec = pl.BlockSpec((1, _D), lambda i: (0, 0))
  return pl.pallas_call(
      _update_body,
      grid=(_N // _BLK,),
      in_specs=[node, part, node, node, node, wspec, bspec],
      out_specs=node,
      out_shape=jax.ShapeDtypeStruct((_N, _D), jnp.float32),
  )(T1a, P4, F, Z, H_prev, W_h, b_h)


def _split_chunks(e):
  total = -(-e // (16 * _CHUNK))
  nc0 = -(-total // 2)
  nc0 += nc0 % 2
  nc1 = max(total - nc0, 2)
  nc1 += nc1 % 2
  return nc0, nc1


def _pack(edge_index, e, nc0, nc1):
  e0 = 16 * nc0 * _CHUNK
  e1 = 16 * nc1 * _CHUNK
  ncp = -(-max(nc0, nc1) // _BLKC) * _BLKC
  ppos = jnp.arange(e0 + e1 - e, dtype=jnp.int32)
  arrs = jnp.stack([edge_index[0], edge_index[1],
                    jnp.arange(e, dtype=jnp.int32)])
  pads = jnp.stack([ppos % _N, _N + ppos % (_ACC_ROWS - _N), ppos % e])
  a = jnp.concatenate([arrs, pads], axis=1)
  a0 = jnp.pad(a[:, :e0].reshape(3, 16, nc0, _CHUNK),
               ((0, 0), (0, 0), (0, ncp - nc0), (0, 0)))
  a1 = jnp.pad(a[:, e0:].reshape(3, 16, nc1, _CHUNK),
               ((0, 0), (0, 0), (0, ncp - nc1), (0, 0)))
  return jnp.concatenate([a0, a1], axis=1)


def kernel(X, H_prev, edge_index, feat, W_z, b_z, W_r, b_r, W_h, b_h):
  e = edge_index.shape[1]
  nc0, nc1 = _split_chunks(e)

  idxs = _pack(edge_index, e, nc0, nc1)

  P1 = _seg_accum(X, idxs, 0, nc0, nc1)
  P2 = _seg_accum(H_prev, idxs, 0, nc0, nc1)
  P3 = _seg_accum(feat, idxs, 2, nc0, nc1)

  Z, M, F, T1a = _gates(P1, P2, P3, H_prev,
                        W_z, W_r, b_z.reshape(1, _D), b_r.reshape(1, _D))

  P4 = _seg_accum(M, idxs, 0, nc0, nc1)

  return _update(T1a, P4, F, Z, H_prev, W_h, b_h.reshape(1, _D))


# speedup vs baseline: 1.2901x; 1.1579x over previous
import functools

import jax
import jax.numpy as jnp
from jax import lax
from jax.experimental import pallas as pl
from jax.experimental.pallas import tpu as pltpu
from jax.experimental.pallas import tpu_sc as plsc

_N = 10000
_D = 128
_NTILES = 32
_STRIPE = 640
_ACC_ROWS = _NTILES // 2 * _STRIPE
_CHUNK = 128
_BLKC = 48
_LANES = 16


def _seg_accum(table, idxs, plane, nc0, nc1):
  mesh = plsc.VectorSubcoreMesh(core_axis_name="c", subcore_axis_name="s")

  def body(table_hbm, idx_hbm, out_hbm, idx_v, dst_v, rows0, rows1,
           acc, sem0, sem1, ssem0, ssem1):
    cid = lax.axis_index("c")
    sid = lax.axis_index("s")
    wid = cid * 16 + sid

    def zrow(i, c):
      r = i // (_D // _LANES)
      k = i % (_D // _LANES)
      rows0[r, pl.ds(k * _LANES, _LANES)] = jnp.zeros((_LANES,), jnp.float32)
      return c
    lax.fori_loop(0, _CHUNK * (_D // _LANES), zrow, 0)
    for j in range(_STRIPE // _CHUNK):
      pltpu.sync_copy(rows0, acc.at[pl.ds(sid * _STRIPE + j * _CHUNK, _CHUNK)])

    plsc.subcore_barrier()

    nc_mine = jnp.where(cid == 0, nc0, nc1)
    nblocks = (nc_mine + _BLKC - 1) // _BLKC

    def gather(j, buf, sem):
      pltpu.async_copy(table_hbm.at[idx_v.at[j]], buf, sem)

    def gather_wait(j, buf, sem):
      pltpu.make_async_copy(table_hbm.at[idx_v.at[j]], buf, sem).wait()

    def scatter(j, buf, sem):
      pltpu.async_copy(buf, acc.at[dst_v.at[j]], sem, add=True)

    def scatter_wait(j, buf, sem):
      pltpu.make_async_copy(buf, acc.at[dst_v.at[j]], sem).wait()

    def block(b, c):
      base = b * _BLKC
      cnt = jnp.minimum(_BLKC, nc_mine - base)
      pltpu.sync_copy(idx_hbm.at[plane, wid, pl.ds(base, _BLKC)], idx_v)
      pltpu.sync_copy(idx_hbm.at[1, wid, pl.ds(base, _BLKC)], dst_v)
      gather(0, rows0, sem0)
      gather(1, rows1, sem1)

      def step2(i, c2):
        j = i * 2
        gather_wait(j, rows0, sem0)
        pltpu.sync_copy(rows0, acc.at[dst_v.at[j]], add=True)
        gather(j + 2, rows0, sem0)

        gather_wait(j + 1, rows1, sem1)
        pltpu.sync_copy(rows1, acc.at[dst_v.at[j + 1]], add=True)
        gather(j + 3, rows1, sem1)
        return c2

      lax.fori_loop(0, cnt // 2 - 1, step2, 0)

      jl = cnt - 2
      gather_wait(jl, rows0, sem0)
      pltpu.sync_copy(rows0, acc.at[dst_v.at[jl]], add=True)
      gather_wait(jl + 1, rows1, sem1)
      pltpu.sync_copy(rows1, acc.at[dst_v.at[jl + 1]], add=True)
      return c

    lax.fori_loop(0, nblocks, block, 0)

    plsc.subcore_barrier()
    pltpu.sync_copy(acc.at[pl.ds(sid * _STRIPE, _STRIPE)],
                    out_hbm.at[cid, pl.ds(sid * _STRIPE, _STRIPE)])

  k = pl.kernel(
      body,
      out_type=jax.ShapeDtypeStruct((2, _ACC_ROWS, _D), jnp.float32),
      mesh=mesh,
      scratch_types=[
          pltpu.VMEM((_BLKC, _CHUNK), jnp.int32),
          pltpu.VMEM((_BLKC, _CHUNK), jnp.int32),
          pltpu.VMEM((_CHUNK, _D), jnp.float32),
          pltpu.VMEM((_CHUNK, _D), jnp.float32),
          pltpu.VMEM_SHARED((_ACC_ROWS, _D), jnp.float32),
          pltpu.SemaphoreType.DMA,
          pltpu.SemaphoreType.DMA,
          pltpu.SemaphoreType.DMA,
          pltpu.SemaphoreType.DMA,
      ],
  )
  return k(table, idxs)


_BLK = 1000

_mm = functools.partial(jnp.dot, preferred_element_type=jnp.float32,
                        precision=lax.Precision.HIGHEST)


def _gates_body(p1, p2, p3, h, wz, wr, bz, br, z_o, m_o, f_o, t1a_o):
  t1a = p1[0] + p1[1]
  t1b = p2[0] + p2[1]
  f = p3[0] + p3[1]
  hh = h[...]
  z = jax.nn.sigmoid(_mm(t1a, wz[:_D]) + _mm(t1b, wz[_D:]) + f + bz[...])
  r = jax.nn.sigmoid(_mm(t1a, wr[:_D]) + _mm(t1b, wr[_D:]) + f + br[...])
  z_o[...] = z
  m_o[...] = r * hh
  f_o[...] = f
  t1a_o[...] = t1a


def _gates(P1, P2, P3, H_prev, W_z, W_r, b_z, b_r):
  part = pl.BlockSpec((2, _BLK, _D), lambda i: (0, i, 0))
  node = pl.BlockSpec((_BLK, _D), lambda i: (i, 0))
  wspec = pl.BlockSpec((2 * _D, _D), lambda i: (0, 0))
  bspec = pl.BlockSpec((1, _D), lambda i: (0, 0))
  out = jax.ShapeDtypeStruct((_N, _D), jnp.float32)
  return pl.pallas_call(
      _gates_body,
      grid=(_N // _BLK,),
      in_specs=[part, part, part, node, wspec, wspec, bspec, bspec],
      out_specs=[node, node, node, node],
      out_shape=[out, out, out, out],
  )(P1, P2, P3, H_prev, W_z, W_r, b_z, b_r)


def _update_body(t1a, p4, f, z, h, wh, bh, h_o):
  t2 = p4[0] + p4[1]
  g = _mm(t1a[...], wh[:_D]) + _mm(t2, wh[_D:]) + f[...] + bh[...]
  h_tilde = jnp.tanh(g)
  zz = z[...]
  h_o[...] = zz * h[...] + (1.0 - zz) * h_tilde


def _update(T1a, P4, F, Z, H_prev, W_h, b_h):
  part = pl.BlockSpec((2, _BLK, _D), lambda i: (0, i, 0))
  node = pl.BlockSpec((_BLK, _D), lambda i: (i, 0))
  wspec = pl.BlockSpec((2 * _D, _D), lambda i: (0, 0))
  bspec = pl.BlockSpec((1, _D), lambda i: (0, 0))
  return pl.pallas_call(
      _update_body,
      grid=(_N // _BLK,),
      in_specs=[node, part, node, node, node, wspec, bspec],
      out_specs=node,
      out_shape=jax.ShapeDtypeStruct((_N, _D), jnp.float32),
  )(T1a, P4, F, Z, H_prev, W_h, b_h)


def _split_chunks(e):
  total = -(-e // (16 * _CHUNK))
  nc0 = -(-total // 2)
  nc0 += nc0 % 2
  nc1 = max(total - nc0, 2)
  nc1 += nc1 % 2
  return nc0, nc1


def _pack(edge_index, e, nc0, nc1):
  e0 = 16 * nc0 * _CHUNK
  e1 = 16 * nc1 * _CHUNK
  ncp = -(-max(nc0, nc1) // _BLKC) * _BLKC
  ppos = jnp.arange(e0 + e1 - e, dtype=jnp.int32)
  arrs = jnp.stack([edge_index[0], edge_index[1],
                    jnp.arange(e, dtype=jnp.int32)])
  pads = jnp.stack([ppos % _N, _N + ppos % (_ACC_ROWS - _N), ppos % e])
  a = jnp.concatenate([arrs, pads], axis=1)
  a0 = jnp.pad(a[:, :e0].reshape(3, 16, nc0, _CHUNK),
               ((0, 0), (0, 0), (0, ncp - nc0), (0, 0)))
  a1 = jnp.pad(a[:, e0:].reshape(3, 16, nc1, _CHUNK),
               ((0, 0), (0, 0), (0, ncp - nc1), (0, 0)))
  return jnp.concatenate([a0, a1], axis=1)


def kernel(X, H_prev, edge_index, feat, W_z, b_z, W_r, b_r, W_h, b_h):
  e = edge_index.shape[1]
  nc0, nc1 = _split_chunks(e)

  idxs = _pack(edge_index, e, nc0, nc1)

  P1 = _seg_accum(X, idxs, 0, nc0, nc1)
  P2 = _seg_accum(H_prev, idxs, 0, nc0, nc1)
  P3 = _seg_accum(feat, idxs, 2, nc0, nc1)

  Z, M, F, T1a = _gates(P1, P2, P3, H_prev,
                        W_z, W_r, b_z.reshape(1, _D), b_r.reshape(1, _D))

  P4 = _seg_accum(M, idxs, 0, nc0, nc1)

  return _update(T1a, P4, F, Z, H_prev, W_h, b_h.reshape(1, _D))


# speedup vs baseline: 1.3728x; 1.0641x over previous
import functools

import jax
import jax.numpy as jnp
from jax import lax
from jax.experimental import pallas as pl
from jax.experimental.pallas import tpu as pltpu
from jax.experimental.pallas import tpu_sc as plsc

_N = 10000
_D = 128
_NTILES = 32
_STRIPE = 632
_ACC_ROWS = _NTILES // 2 * _STRIPE
_CHUNK = 96
_BLKC = 48
_LANES = 16


def _seg_accum(table, idxs, plane, nc0, nc1):
  mesh = plsc.VectorSubcoreMesh(core_axis_name="c", subcore_axis_name="s")

  def body(table_hbm, idx_hbm, out_hbm, idx_v, dst_v, rows0, rows1, rows2,
           acc, sem0, sem1, sem2):
    cid = lax.axis_index("c")
    sid = lax.axis_index("s")
    wid = cid * 16 + sid

    def zrow(i, c):
      r = i // (_D // _LANES)
      k = i % (_D // _LANES)
      rows0[r, pl.ds(k * _LANES, _LANES)] = jnp.zeros((_LANES,), jnp.float32)
      return c
    lax.fori_loop(0, _CHUNK * (_D // _LANES), zrow, 0)
    for j in range(_STRIPE // _CHUNK):
      pltpu.sync_copy(rows0, acc.at[pl.ds(sid * _STRIPE + j * _CHUNK, _CHUNK)])
    rem = _STRIPE % _CHUNK
    if rem:
      pltpu.sync_copy(
          rows0.at[pl.ds(0, rem)],
          acc.at[pl.ds(sid * _STRIPE + (_STRIPE // _CHUNK) * _CHUNK, rem)])

    plsc.subcore_barrier()

    nc_mine = jnp.where(cid == 0, nc0, nc1)
    nblocks = (nc_mine + _BLKC - 1) // _BLKC

    def gather(j, buf, sem):
      pltpu.async_copy(table_hbm.at[idx_v.at[j]], buf, sem)

    def gather_wait(j, buf, sem):
      pltpu.make_async_copy(table_hbm.at[idx_v.at[j]], buf, sem).wait()

    def scatter(j, buf, sem):
      pltpu.async_copy(buf, acc.at[dst_v.at[j]], sem, add=True)

    def scatter_wait(j, buf, sem):
      pltpu.make_async_copy(buf, acc.at[dst_v.at[j]], sem).wait()

    def block(b, c):
      base = b * _BLKC
      cnt = jnp.minimum(_BLKC, nc_mine - base)
      pltpu.sync_copy(idx_hbm.at[plane, wid, pl.ds(base, _BLKC)], idx_v)
      pltpu.sync_copy(idx_hbm.at[1, wid, pl.ds(base, _BLKC)], dst_v)
      gather(0, rows0, sem0)
      gather(1, rows1, sem1)
      gather(2, rows2, sem2)

      def step3(i, c2):
        j = i * 3
        gather_wait(j, rows0, sem0)
        pltpu.sync_copy(rows0, acc.at[dst_v.at[j]], add=True)
        gather(j + 3, rows0, sem0)

        gather_wait(j + 1, rows1, sem1)
        pltpu.sync_copy(rows1, acc.at[dst_v.at[j + 1]], add=True)
        gather(j + 4, rows1, sem1)

        gather_wait(j + 2, rows2, sem2)
        pltpu.sync_copy(rows2, acc.at[dst_v.at[j + 2]], add=True)
        gather(j + 5, rows2, sem2)
        return c2

      lax.fori_loop(0, cnt // 3 - 1, step3, 0)

      jl = cnt - 3
      gather_wait(jl, rows0, sem0)
      pltpu.sync_copy(rows0, acc.at[dst_v.at[jl]], add=True)
      gather_wait(jl + 1, rows1, sem1)
      pltpu.sync_copy(rows1, acc.at[dst_v.at[jl + 1]], add=True)
      gather_wait(jl + 2, rows2, sem2)
      pltpu.sync_copy(rows2, acc.at[dst_v.at[jl + 2]], add=True)
      return c

    lax.fori_loop(0, nblocks, block, 0)

    plsc.subcore_barrier()
    pltpu.sync_copy(acc.at[pl.ds(sid * _STRIPE, _STRIPE)],
                    out_hbm.at[cid, pl.ds(sid * _STRIPE, _STRIPE)])

  k = pl.kernel(
      body,
      out_type=jax.ShapeDtypeStruct((2, _ACC_ROWS, _D), jnp.float32),
      mesh=mesh,
      scratch_types=[
          pltpu.VMEM((_BLKC, _CHUNK), jnp.int32),
          pltpu.VMEM((_BLKC, _CHUNK), jnp.int32),
          pltpu.VMEM((_CHUNK, _D), jnp.float32),
          pltpu.VMEM((_CHUNK, _D), jnp.float32),
          pltpu.VMEM((_CHUNK, _D), jnp.float32),
          pltpu.VMEM_SHARED((_ACC_ROWS, _D), jnp.float32),
          pltpu.SemaphoreType.DMA,
          pltpu.SemaphoreType.DMA,
          pltpu.SemaphoreType.DMA,
      ],
  )
  return k(table, idxs)


_BLK = 1000

_mm = functools.partial(jnp.dot, preferred_element_type=jnp.float32,
                        precision=lax.Precision.HIGHEST)


def _gates_body(p1, p2, p3, h, wz, wr, bz, br, z_o, m_o, f_o, t1a_o):
  t1a = p1[0] + p1[1]
  t1b = p2[0] + p2[1]
  f = p3[0] + p3[1]
  hh = h[...]
  z = jax.nn.sigmoid(_mm(t1a, wz[:_D]) + _mm(t1b, wz[_D:]) + f + bz[...])
  r = jax.nn.sigmoid(_mm(t1a, wr[:_D]) + _mm(t1b, wr[_D:]) + f + br[...])
  z_o[...] = z
  m_o[...] = r * hh
  f_o[...] = f
  t1a_o[...] = t1a


def _gates(P1, P2, P3, H_prev, W_z, W_r, b_z, b_r):
  part = pl.BlockSpec((2, _BLK, _D), lambda i: (0, i, 0))
  node = pl.BlockSpec((_BLK, _D), lambda i: (i, 0))
  wspec = pl.BlockSpec((2 * _D, _D), lambda i: (0, 0))
  bspec = pl.BlockSpec((1, _D), lambda i: (0, 0))
  out = jax.ShapeDtypeStruct((_N, _D), jnp.float32)
  return pl.pallas_call(
      _gates_body,
      grid=(_N // _BLK,),
      in_specs=[part, part, part, node, wspec, wspec, bspec, bspec],
      out_specs=[node, node, node, node],
      out_shape=[out, out, out, out],
  )(P1, P2, P3, H_prev, W_z, W_r, b_z, b_r)


def _update_body(t1a, p4, f, z, h, wh, bh, h_o):
  t2 = p4[0] + p4[1]
  g = _mm(t1a[...], wh[:_D]) + _mm(t2, wh[_D:]) + f[...] + bh[...]
  h_tilde = jnp.tanh(g)
  zz = z[...]
  h_o[...] = zz * h[...] + (1.0 - zz) * h_tilde


def _update(T1a, P4, F, Z, H_prev, W_h, b_h):
  part = pl.BlockSpec((2, _BLK, _D), lambda i: (0, i, 0))
  node = pl.BlockSpec((_BLK, _D), lambda i: (i, 0))
  wspec = pl.BlockSpec((2 * _D, _D), lambda i: (0, 0))
  bspec = pl.BlockSpec((1, _D), lambda i: (0, 0))
  return pl.pallas_call(
      _update_body,
      grid=(_N // _BLK,),
      in_specs=[node, part, node, node, node, wspec, bspec],
      out_specs=node,
      out_shape=jax.ShapeDtypeStruct((_N, _D), jnp.float32),
  )(T1a, P4, F, Z, H_prev, W_h, b_h)


def _split_chunks(e):
  total = -(-e // (16 * _CHUNK))
  nc0 = -(-(-(-total // 2)) // 3) * 3
  nc1 = -(-max(total - nc0, 3) // 3) * 3
  return nc0, nc1


def _pack(edge_index, e, nc0, nc1):
  e0 = 16 * nc0 * _CHUNK
  e1 = 16 * nc1 * _CHUNK
  ncp = -(-max(nc0, nc1) // _BLKC) * _BLKC
  ppos = jnp.arange(e0 + e1 - e, dtype=jnp.int32)
  arrs = jnp.stack([edge_index[0], edge_index[1],
                    jnp.arange(e, dtype=jnp.int32)])
  pads = jnp.stack([ppos % _N, _N + ppos % (_ACC_ROWS - _N), ppos % e])
  a = jnp.concatenate([arrs, pads], axis=1)
  a0 = jnp.pad(a[:, :e0].reshape(3, 16, nc0, _CHUNK),
               ((0, 0), (0, 0), (0, ncp - nc0), (0, 0)))
  a1 = jnp.pad(a[:, e0:].reshape(3, 16, nc1, _CHUNK),
               ((0, 0), (0, 0), (0, ncp - nc1), (0, 0)))
  return jnp.concatenate([a0, a1], axis=1)


def kernel(X, H_prev, edge_index, feat, W_z, b_z, W_r, b_r, W_h, b_h):
  e = edge_index.shape[1]
  nc0, nc1 = _split_chunks(e)

  idxs = _pack(edge_index, e, nc0, nc1)

  P1 = _seg_accum(X, idxs, 0, nc0, nc1)
  P2 = _seg_accum(H_prev, idxs, 0, nc0, nc1)
  P3 = _seg_accum(feat, idxs, 2, nc0, nc1)

  Z, M, F, T1a = _gates(P1, P2, P3, H_prev,
                        W_z, W_r, b_z.reshape(1, _D), b_r.reshape(1, _D))

  P4 = _seg_accum(M, idxs, 0, nc0, nc1)

  return _update(T1a, P4, F, Z, H_prev, W_h, b_h.reshape(1, _D))


# speedup vs baseline: 1.4263x; 1.0389x over previous
import functools

import jax
import jax.numpy as jnp
from jax import lax
from jax.experimental import pallas as pl
from jax.experimental.pallas import tpu as pltpu
from jax.experimental.pallas import tpu_sc as plsc

_N = 10000
_D = 128
_NTILES = 32
_STRIPE = 632
_ACC_ROWS = _NTILES // 2 * _STRIPE
_CHUNK = 96
_BLKC = 48
_LANES = 16


def _seg_accum(table, idxs, plane, nc0, nc1):
  mesh = plsc.VectorSubcoreMesh(core_axis_name="c", subcore_axis_name="s")

  def body(table_hbm, idx_hbm, out_hbm, idx_v, dst_v, rows0, rows1, rows2,
           acc, sem0, sem1, sem2):
    cid = lax.axis_index("c")
    sid = lax.axis_index("s")
    wid = cid * 16 + sid

    def zrow(i, c):
      r = i // (_D // _LANES)
      k = i % (_D // _LANES)
      rows0[r, pl.ds(k * _LANES, _LANES)] = jnp.zeros((_LANES,), jnp.float32)
      return c
    lax.fori_loop(0, _CHUNK * (_D // _LANES), zrow, 0)
    for j in range(_STRIPE // _CHUNK):
      pltpu.sync_copy(rows0, acc.at[pl.ds(sid * _STRIPE + j * _CHUNK, _CHUNK)])
    rem = _STRIPE % _CHUNK
    if rem:
      pltpu.sync_copy(
          rows0.at[pl.ds(0, rem)],
          acc.at[pl.ds(sid * _STRIPE + (_STRIPE // _CHUNK) * _CHUNK, rem)])

    plsc.subcore_barrier()

    nc_mine = jnp.where(cid == 0, nc0, nc1)
    nblocks = (nc_mine + _BLKC - 1) // _BLKC

    def gather(j, buf, sem):
      pltpu.async_copy(table_hbm.at[idx_v.at[j]], buf, sem)

    def gather_wait(j, buf, sem):
      pltpu.make_async_copy(table_hbm.at[idx_v.at[j]], buf, sem).wait()

    def scatter(j, buf, sem):
      pltpu.async_copy(buf, acc.at[dst_v.at[j]], sem, add=True)

    def scatter_wait(j, buf, sem):
      pltpu.make_async_copy(buf, acc.at[dst_v.at[j]], sem).wait()

    def block(b, c):
      base = b * _BLKC
      cnt = jnp.minimum(_BLKC, nc_mine - base)
      pltpu.sync_copy(idx_hbm.at[plane, wid, pl.ds(base, _BLKC)], idx_v)
      pltpu.sync_copy(idx_hbm.at[1, wid, pl.ds(base, _BLKC)], dst_v)
      gather(0, rows0, sem0)
      gather(1, rows1, sem1)
      gather(2, rows2, sem2)

      def step3(i, c2):
        j = i * 3
        gather_wait(j, rows0, sem0)
        pltpu.sync_copy(rows0, acc.at[dst_v.at[j]], add=True)
        gather(j + 3, rows0, sem0)

        gather_wait(j + 1, rows1, sem1)
        pltpu.sync_copy(rows1, acc.at[dst_v.at[j + 1]], add=True)
        gather(j + 4, rows1, sem1)

        gather_wait(j + 2, rows2, sem2)
        pltpu.sync_copy(rows2, acc.at[dst_v.at[j + 2]], add=True)
        gather(j + 5, rows2, sem2)
        return c2

      lax.fori_loop(0, cnt // 3 - 1, step3, 0)

      jl = cnt - 3
      gather_wait(jl, rows0, sem0)
      pltpu.sync_copy(rows0, acc.at[dst_v.at[jl]], add=True)
      gather_wait(jl + 1, rows1, sem1)
      pltpu.sync_copy(rows1, acc.at[dst_v.at[jl + 1]], add=True)
      gather_wait(jl + 2, rows2, sem2)
      pltpu.sync_copy(rows2, acc.at[dst_v.at[jl + 2]], add=True)
      return c

    lax.fori_loop(0, nblocks, block, 0)

    plsc.subcore_barrier()
    pltpu.sync_copy(acc.at[pl.ds(sid * _STRIPE, _STRIPE)],
                    out_hbm.at[cid, pl.ds(sid * _STRIPE, _STRIPE)])

  k = pl.kernel(
      body,
      out_type=jax.ShapeDtypeStruct((2, _ACC_ROWS, _D), jnp.float32),
      mesh=mesh,
      scratch_types=[
          pltpu.VMEM((_BLKC, _CHUNK), jnp.int32),
          pltpu.VMEM((_BLKC, _CHUNK), jnp.int32),
          pltpu.VMEM((_CHUNK, _D), jnp.float32),
          pltpu.VMEM((_CHUNK, _D), jnp.float32),
          pltpu.VMEM((_CHUNK, _D), jnp.float32),
          pltpu.VMEM_SHARED((_ACC_ROWS, _D), jnp.float32),
          pltpu.SemaphoreType.DMA,
          pltpu.SemaphoreType.DMA,
          pltpu.SemaphoreType.DMA,
      ],
  )
  return k(table, idxs)


_BLK = 2000

_mm = functools.partial(jnp.dot, preferred_element_type=jnp.float32,
                        precision=lax.Precision.HIGHEST)


def _gates_body(p1, p2, p3, h, wz, wr, bz, br, z_o, m_o, f_o, t1a_o):
  t1a = p1[0] + p1[1]
  t1b = p2[0] + p2[1]
  f = p3[0] + p3[1]
  hh = h[...]
  z = jax.nn.sigmoid(_mm(t1a, wz[:_D]) + _mm(t1b, wz[_D:]) + f + bz[...])
  r = jax.nn.sigmoid(_mm(t1a, wr[:_D]) + _mm(t1b, wr[_D:]) + f + br[...])
  z_o[...] = z
  m_o[...] = r * hh
  f_o[...] = f
  t1a_o[...] = t1a


def _gates(P1, P2, P3, H_prev, W_z, W_r, b_z, b_r):
  part = pl.BlockSpec((2, _BLK, _D), lambda i: (0, i, 0))
  node = pl.BlockSpec((_BLK, _D), lambda i: (i, 0))
  wspec = pl.BlockSpec((2 * _D, _D), lambda i: (0, 0))
  bspec = pl.BlockSpec((1, _D), lambda i: (0, 0))
  out = jax.ShapeDtypeStruct((_N, _D), jnp.float32)
  return pl.pallas_call(
      _gates_body,
      grid=(_N // _BLK,),
      in_specs=[part, part, part, node, wspec, wspec, bspec, bspec],
      out_specs=[node, node, node, node],
      out_shape=[out, out, out, out],
  )(P1, P2, P3, H_prev, W_z, W_r, b_z, b_r)


def _update_body(t1a, p4, f, z, h, wh, bh, h_o):
  t2 = p4[0] + p4[1]
  g = _mm(t1a[...], wh[:_D]) + _mm(t2, wh[_D:]) + f[...] + bh[...]
  h_tilde = jnp.tanh(g)
  zz = z[...]
  h_o[...] = zz * h[...] + (1.0 - zz) * h_tilde


def _update(T1a, P4, F, Z, H_prev, W_h, b_h):
  part = pl.BlockSpec((2, _BLK, _D), lambda i: (0, i, 0))
  node = pl.BlockSpec((_BLK, _D), lambda i: (i, 0))
  wspec = pl.BlockSpec((2 * _D, _D), lambda i: (0, 0))
  bspec = pl.BlockSpec((1, _D), lambda i: (0, 0))
  return pl.pallas_call(
      _update_body,
      grid=(_N // _BLK,),
      in_specs=[node, part, node, node, node, wspec, bspec],
      out_specs=node,
      out_shape=jax.ShapeDtypeStruct((_N, _D), jnp.float32),
  )(T1a, P4, F, Z, H_prev, W_h, b_h)


def _split_chunks(e):
  total = -(-e // (16 * _CHUNK))
  nc0 = -(-(-(-total // 2)) // 3) * 3
  nc1 = -(-max(total - nc0, 3) // 3) * 3
  return nc0, nc1


def _pack(edge_index, e, nc0, nc1):
  e0 = 16 * nc0 * _CHUNK
  e1 = 16 * nc1 * _CHUNK
  ncp = -(-max(nc0, nc1) // _BLKC) * _BLKC
  ppos = jnp.arange(e0 + e1 - e, dtype=jnp.int32)
  arrs = jnp.stack([edge_index[0], edge_index[1],
                    jnp.arange(e, dtype=jnp.int32)])
  pads = jnp.stack([ppos % _N, _N + ppos % (_ACC_ROWS - _N), ppos % e])
  a = jnp.concatenate([arrs, pads], axis=1)
  a0 = jnp.pad(a[:, :e0].reshape(3, 16, nc0, _CHUNK),
               ((0, 0), (0, 0), (0, ncp - nc0), (0, 0)))
  a1 = jnp.pad(a[:, e0:].reshape(3, 16, nc1, _CHUNK),
               ((0, 0), (0, 0), (0, ncp - nc1), (0, 0)))
  return jnp.concatenate([a0, a1], axis=1)


def kernel(X, H_prev, edge_index, feat, W_z, b_z, W_r, b_r, W_h, b_h):
  e = edge_index.shape[1]
  nc0, nc1 = _split_chunks(e)

  idxs = _pack(edge_index, e, nc0, nc1)

  P1 = _seg_accum(X, idxs, 0, nc0, nc1)
  P2 = _seg_accum(H_prev, idxs, 0, nc0, nc1)
  P3 = _seg_accum(feat, idxs, 2, nc0, nc1)

  Z, M, F, T1a = _gates(P1, P2, P3, H_prev,
                        W_z, W_r, b_z.reshape(1, _D), b_r.reshape(1, _D))

  P4 = _seg_accum(M, idxs, 0, nc0, nc1)

  return _update(T1a, P4, F, Z, H_prev, W_h, b_h.reshape(1, _D))
